# baseline (speedup 1.0000x reference)
import functools

import jax
import jax.numpy as jnp
from jax import lax
from jax.experimental import pallas as pl
from jax.experimental.pallas import tpu as pltpu
from jax.experimental.pallas import tpu_sc as plsc

B = 4096
D = 32
DP = 128
NREL = 64
DD = D * D
NW = 32
RBLK = 512
NBLK = B // RBLK
NROWS = 1000000
NSLAB = NROWS // 128
TAILBASE = NSLAB * 128
CW = 512
LCAP = B + 16
OUTROWS = 8 * B + RBLK
TRASH = 8 * B


def _sc_body(users_h, pos_h, neg_h, heads_h, pt_h, nt_h, uT, iT, kT, out_h,
             lr_u, ld_u, lr_p, ld_p, lr_n, ld_n,
             lr_h, ld_h, lr_pt, ld_pt, lr_nt, ld_nt,
             cr, cd, istage, cbuf, stage, didx, sem):
    c = lax.axis_index("c")
    s = lax.axis_index("s")
    wid = s * 2 + c
    s0 = (wid * NSLAB) // NW
    s1 = ((wid + 1) * NSLAB) // NW
    col0 = s0 * 128
    ncols = (s1 - s0) * 128
    iota16 = lax.broadcasted_iota(jnp.int32, (16,), 0)

    def prefill_didx():
        for q in range(4):
            didx[pl.ds(q * 16, 16)] = jnp.full((16,), TRASH, jnp.int32)

    prefill_didx()

    def build(idx_h, lr, ld):
        def piece(p, cnt):
            pltpu.sync_copy(idx_h.at[pl.ds(p * 2048, 2048)], istage)

            def step(i, cnt):
                r = istage[pl.ds(i * 16, 16)]
                d = p * 2048 + i * 16 + iota16
                m = (r >= col0) & (r < col0 + ncols)
                pos = cnt + plsc.cumsum(m.astype(jnp.int32)) - 1
                plsc.store_scatter(lr, [pos], r, mask=m)
                plsc.store_scatter(ld, [pos], d, mask=m)
                return cnt + jnp.sum(m.astype(jnp.int32))

            return lax.fori_loop(0, 2048 // 16, step, cnt)

        return lax.fori_loop(0, B // 2048, piece, 0)

    n_u = build(users_h, lr_u, ld_u)
    n_p = build(pos_h, lr_p, ld_p)
    n_n = build(neg_h, lr_n, ld_n)
    n_h = build(heads_h, lr_h, ld_h)
    n_pt = build(pt_h, lr_pt, ld_pt)
    n_nt = build(nt_h, lr_nt, ld_nt)

    def flush():
        pltpu.async_copy(stage, out_h.at[didx], sem).wait()
        prefill_didx()

    def scan_pair(lr, ld, nL, pair_off, ccol0, w, cnt):
        def sstep(i, mcnt):
            valid = (i * 16 + iota16) < nL
            r = lr[pl.ds(i * 16, 16)]
            d = ld[pl.ds(i * 16, 16)]
            m = valid & (r >= ccol0) & (r < ccol0 + w)
            pos = mcnt + plsc.cumsum(m.astype(jnp.int32)) - 1
            plsc.store_scatter(cr, [pos], r, mask=m)
            plsc.store_scatter(cd, [pos], d + pair_off * B, mask=m)
            return mcnt + jnp.sum(m.astype(jnp.int32))

        m = lax.fori_loop(0, (nL + 15) // 16, sstep, 0)

        def grp(gi, cnt):
            full = cnt > 48

            @pl.when(full)
            def _():
                flush()

            cnt = jnp.where(full, 0, cnt)
            v = (gi * 16 + iota16) < m
            colv = cr[pl.ds(gi * 16, 16)] - ccol0
            dv = cd[pl.ds(gi * 16, 16)]
            rows = cnt + iota16

            def feat(f, carry):
                fv = jnp.zeros((16,), jnp.int32) + f
                vals = plsc.load_gather(cbuf, [fv, colv], mask=v)
                plsc.store_scatter(stage, [rows, fv], vals, mask=v)
                return carry

            lax.fori_loop(0, D, feat, 0)
            plsc.store_scatter(didx, [rows], dv, mask=v)
            return cnt + 16

        return lax.fori_loop(0, (m + 15) // 16, grp, cnt)

    tables = ((uT, ((lr_u, ld_u, n_u, 0),)),
              (iT, ((lr_p, ld_p, n_p, 1), (lr_n, ld_n, n_n, 3))),
              (kT, ((lr_p, ld_p, n_p, 2), (lr_n, ld_n, n_n, 4),
                    (lr_h, ld_h, n_h, 5), (lr_pt, ld_pt, n_pt, 6),
                    (lr_nt, ld_nt, n_nt, 7))))
    for tbl, pairs in tables:
        nfull = ncols // CW
        nrem = (ncols - nfull * CW) // 128

        def chunk(ci, cnt, tbl=tbl, pairs=pairs):
            ccol0 = col0 + ci * CW
            pltpu.sync_copy(tbl.at[:, pl.ds(ccol0, CW)], cbuf)
            for lr, ld, nL, off in pairs:
                cnt = scan_pair(lr, ld, nL, off, ccol0, CW, cnt)
            return cnt

        cnt = lax.fori_loop(0, nfull, chunk, 0)

        def rchunk(ri, cnt, tbl=tbl, pairs=pairs, nfull=nfull):
            ccol0 = col0 + nfull * CW + ri * 128
            pltpu.sync_copy(tbl.at[:, pl.ds(ccol0, 128)],
                            cbuf.at[:, pl.ds(0, 128)])
            for lr, ld, nL, off in pairs:
                cnt = scan_pair(lr, ld, nL, off, ccol0, 128, cnt)
            return cnt

        cnt = lax.fori_loop(0, nrem, rchunk, cnt)
        flush()


def _sc_gather(users, pos_items, neg_items, heads, pos_tails, neg_tails,
               user_T, item_T, kg_T):
    mesh = plsc.VectorSubcoreMesh(core_axis_name="c", subcore_axis_name="s")
    f = pl.kernel(
        _sc_body,
        out_type=jax.ShapeDtypeStruct((OUTROWS, DP), jnp.float32),
        mesh=mesh,
        scratch_types=(
            [pltpu.VMEM((LCAP,), jnp.int32)] * 12
            + [pltpu.VMEM((LCAP,), jnp.int32)] * 2
            + [pltpu.VMEM((2048,), jnp.int32)]
            + [pltpu.VMEM((D, CW), jnp.float32)]
            + [pltpu.VMEM((64, DP), jnp.float32)]
            + [pltpu.VMEM((64,), jnp.int32)]
            + [pltpu.SemaphoreType.DMA]
        ),
        compiler_params=pltpu.CompilerParams(use_tc_tiling_on_sc=True,
                                             needs_layout_passes=False),
    )
    return f(users, pos_items, neg_items, heads, pos_tails, neg_tails,
             user_T, item_T, kg_T)


def _l2n(x):
    n = jnp.sqrt(jnp.sum(x * x, axis=1, keepdims=True))
    return x / jnp.maximum(n, 1e-12)


def _rowwork_body(rel_ref, iu_ref, ip_ref, in_ref, ih_ref, ipt_ref, int_ref,
                  gu_ref, gpi_ref, gpik_ref, gni_ref, gnik_ref,
                  gh_ref, gpt_ref, gnt_ref,
                  tu_ref, ti_ref, tk_ref, rel_emb_ref, wflat_ref,
                  ue_ref, picomb_ref, nicomb_ref, he_ref, re_ref, pte_ref,
                  nte_ref):
    def fix(idx_ref, g_ref, tail_ref):
        idx2 = idx_ref[0, 0, :][:, None]
        tm2 = idx2 >= TAILBASE
        oh = ((idx2 - TAILBASE)
              == lax.broadcasted_iota(jnp.int32, (RBLK, 64), 1))
        ohf = (oh & tm2).astype(jnp.float32)
        tv = jnp.dot(ohf, tail_ref[...], preferred_element_type=jnp.float32)
        tm2d = jnp.broadcast_to(tm2, (RBLK, D))
        return jnp.where(tm2d, tv, g_ref[:, :D])

    u_e = fix(iu_ref, gu_ref, tu_ref)
    pi = fix(ip_ref, gpi_ref, ti_ref)
    pik = fix(ip_ref, gpik_ref, tk_ref)
    ni = fix(in_ref, gni_ref, ti_ref)
    nik = fix(in_ref, gnik_ref, tk_ref)
    h = fix(ih_ref, gh_ref, tk_ref)
    pt = fix(ipt_ref, gpt_ref, tk_ref)
    nt = fix(int_ref, gnt_ref, tk_ref)

    rel = rel_ref[0, 0, :]
    onehot = (rel[:, None] == lax.broadcasted_iota(jnp.int32, (RBLK, NREL), 1)
              ).astype(jnp.float32)
    re_ref[...] = _l2n(jnp.dot(onehot, rel_emb_ref[...],
                               preferred_element_type=jnp.float32))
    wg = jnp.dot(onehot, wflat_ref[...], preferred_element_type=jnp.float32)
    R = (lax.broadcasted_iota(jnp.int32, (D, DD), 1) % D
         == lax.broadcasted_iota(jnp.int32, (D, DD), 0)).astype(jnp.float32)
    S = (lax.broadcasted_iota(jnp.int32, (DD, D), 0) // D
         == lax.broadcasted_iota(jnp.int32, (DD, D), 1)).astype(jnp.float32)
    for x, o_ref in ((h, he_ref), (pt, pte_ref), (nt, nte_ref)):
        xt = jnp.dot(x, R, preferred_element_type=jnp.float32)
        proj = jnp.dot(xt * wg, S, preferred_element_type=jnp.float32)
        o_ref[...] = _l2n(proj)
    ue_ref[...] = u_e
    picomb_ref[...] = pi + pik
    nicomb_ref[...] = ni + nik


def _matmul_body(u_ref, c_ref, o_ref):
    o_ref[...] = lax.dot_general(u_ref[...], c_ref[...],
                                 (((1,), (1,)), ((), ())),
                                 preferred_element_type=jnp.float32)


def kernel(users, pos_items, neg_items, heads, relations, pos_tails, neg_tails,
           user_embed, item_embed, kg_entity_embed, kg_relation_embed, trans_W):
    packed = _sc_gather(users, pos_items, neg_items, heads, pos_tails,
                        neg_tails, user_embed.T, item_embed.T,
                        kg_entity_embed.T)
    tails_u = user_embed[TAILBASE:]
    tails_i = item_embed[TAILBASE:]
    tails_k = kg_entity_embed[TAILBASE:]
    wflat = trans_W.transpose(0, 2, 1).reshape(NREL, DD)
    rel3 = relations.reshape(NBLK, 1, RBLK)
    idx3 = [a.reshape(NBLK, 1, RBLK) for a in
            (users, pos_items, neg_items, heads, pos_tails, neg_tails)]
    idx_spec = pl.BlockSpec((1, 1, RBLK), lambda i: (i, 0, 0))

    def packed_spec(k):
        return pl.BlockSpec((RBLK, DP), lambda i, k=k: (k * NBLK + i, 0))

    out_spec = pl.BlockSpec((RBLK, D), lambda i: (i, 0))
    u_e, picomb, nicomb, h_e, r_e, pt_e, nt_e = pl.pallas_call(
        _rowwork_body,
        grid=(NBLK,),
        in_specs=[idx_spec] * 7
        + [packed_spec(k) for k in range(8)]
        + [pl.BlockSpec((64, D), lambda i: (0, 0))] * 3
        + [pl.BlockSpec((NREL, D), lambda i: (0, 0)),
           pl.BlockSpec((NREL, DD), lambda i: (0, 0))],
        out_specs=[out_spec] * 7,
        out_shape=[jax.ShapeDtypeStruct((B, D), jnp.float32)] * 7,
    )(rel3, *idx3, *([packed] * 8), tails_u, tails_i, tails_k,
      kg_relation_embed, wflat)
    preds = pl.pallas_call(
        _matmul_body,
        grid=(NBLK,),
        in_specs=[pl.BlockSpec((RBLK, D), lambda i: (i, 0)),
                  pl.BlockSpec((B, D), lambda i: (0, 0))],
        out_specs=pl.BlockSpec((RBLK, B), lambda i: (i, 0)),
        out_shape=jax.ShapeDtypeStruct((B, B), jnp.float32),
    )(u_e, picomb)
    return (u_e, picomb, nicomb, h_e, r_e, pt_e, nt_e, preds)


# speedup vs baseline: 5.8923x; 5.8923x over previous
import functools

import jax
import jax.numpy as jnp
from jax import lax
from jax.experimental import pallas as pl
from jax.experimental.pallas import tpu as pltpu
from jax.experimental.pallas import tpu_sc as plsc

B = 4096
D = 32
NREL = 64
DD = D * D
NW = 32
BPW = B // NW
RBLK = 512
NBLK = B // RBLK


def _sc_gather_body(users_h, pos_h, neg_h, heads_h, pt_h, nt_h,
                    ue_h, ie_h, ke_h,
                    o_u, o_pi, o_pik, o_ni, o_nik, o_h, o_pt, o_nt,
                    iu, ip, ineg, ih, ipt, int_,
                    r0, r1, r2, r3, r4, r5, r6, r7, sem):
    c = lax.axis_index("c")
    s = lax.axis_index("s")
    wid = s * 2 + c
    base = wid * BPW
    for hb, vb in ((users_h, iu), (pos_h, ip), (neg_h, ineg),
                   (heads_h, ih), (pt_h, ipt), (nt_h, int_)):
        pltpu.sync_copy(hb.at[pl.ds(base, BPW)], vb)
    gathers = ((ue_h, iu, r0), (ie_h, ip, r1), (ke_h, ip, r2),
               (ie_h, ineg, r3), (ke_h, ineg, r4),
               (ke_h, ih, r5), (ke_h, ipt, r6), (ke_h, int_, r7))
    copies = [pltpu.async_copy(tbl.at[vb], rv, sem) for tbl, vb, rv in gathers]
    for cp in copies:
        cp.wait()
    for rv, oh in zip((r0, r1, r2, r3, r4, r5, r6, r7),
                      (o_u, o_pi, o_pik, o_ni, o_nik, o_h, o_pt, o_nt)):
        pltpu.sync_copy(rv, oh.at[pl.ds(base, BPW)])


def _sc_gather(users, pos_items, neg_items, heads, pos_tails, neg_tails,
               user_embed, item_embed, kg_entity_embed):
    mesh = plsc.VectorSubcoreMesh(core_axis_name="c", subcore_axis_name="s")
    f = pl.kernel(
        _sc_gather_body,
        out_type=[jax.ShapeDtypeStruct((B, D), jnp.float32)] * 8,
        mesh=mesh,
        scratch_types=(
            [pltpu.VMEM((BPW,), jnp.int32)] * 6
            + [pltpu.VMEM((BPW, D), jnp.float32)] * 8
            + [pltpu.SemaphoreType.DMA]
        ),
        compiler_params=pltpu.CompilerParams(use_tc_tiling_on_sc=False),
    )
    return f(users, pos_items, neg_items, heads, pos_tails, neg_tails,
             user_embed, item_embed, kg_entity_embed)


def _l2n(x):
    n = jnp.sqrt(jnp.sum(x * x, axis=1, keepdims=True))
    return x / jnp.maximum(n, 1e-12)


def _rowwork_body(rel_ref, pie_ref, pik_ref, nie_ref, nik_ref,
                  h_ref, pt_ref, nt_ref, rel_emb_ref, wflat_ref,
                  picomb_ref, nicomb_ref, he_ref, re_ref, pte_ref, nte_ref):
    rel = rel_ref[0, 0, :]
    onehot = (rel[:, None] == lax.broadcasted_iota(jnp.int32, (RBLK, NREL), 1)
              ).astype(jnp.float32)
    re_ref[...] = _l2n(jnp.dot(onehot, rel_emb_ref[...],
                               preferred_element_type=jnp.float32))
    wg = jnp.dot(onehot, wflat_ref[...], preferred_element_type=jnp.float32)
    R = (lax.broadcasted_iota(jnp.int32, (D, DD), 1) % D
         == lax.broadcasted_iota(jnp.int32, (D, DD), 0)).astype(jnp.float32)
    S = (lax.broadcasted_iota(jnp.int32, (DD, D), 0) // D
         == lax.broadcasted_iota(jnp.int32, (DD, D), 1)).astype(jnp.float32)
    for x_ref, o_ref in ((h_ref, he_ref), (pt_ref, pte_ref), (nt_ref, nte_ref)):
        xt = jnp.dot(x_ref[...], R, preferred_element_type=jnp.float32)
        proj = jnp.dot(xt * wg, S, preferred_element_type=jnp.float32)
        o_ref[...] = _l2n(proj)
    picomb_ref[...] = pie_ref[...] + pik_ref[...]
    nicomb_ref[...] = nie_ref[...] + nik_ref[...]


def _matmul_body(u_ref, c_ref, o_ref):
    o_ref[...] = lax.dot_general(u_ref[...], c_ref[...],
                                 (((1,), (1,)), ((), ())),
                                 preferred_element_type=jnp.float32)


def kernel(users, pos_items, neg_items, heads, relations, pos_tails, neg_tails,
           user_embed, item_embed, kg_entity_embed, kg_relation_embed, trans_W):
    u_e, pie, pik, nie, nik, h_raw, pt_raw, nt_raw = _sc_gather(
        users, pos_items, neg_items, heads, pos_tails, neg_tails,
        user_embed, item_embed, kg_entity_embed)
    wflat = trans_W.transpose(0, 2, 1).reshape(NREL, DD)
    rel3 = relations.reshape(NBLK, 1, RBLK)
    row_spec = pl.BlockSpec((RBLK, D), lambda i: (i, 0))
    picomb, nicomb, h_e, r_e, pt_e, nt_e = pl.pallas_call(
        _rowwork_body,
        grid=(NBLK,),
        in_specs=[pl.BlockSpec((1, 1, RBLK), lambda i: (i, 0, 0))]
        + [row_spec] * 7
        + [pl.BlockSpec((NREL, D), lambda i: (0, 0)),
           pl.BlockSpec((NREL, DD), lambda i: (0, 0))],
        out_specs=[row_spec] * 6,
        out_shape=[jax.ShapeDtypeStruct((B, D), jnp.float32)] * 6,
    )(rel3, pie, pik, nie, nik, h_raw, pt_raw, nt_raw,
      kg_relation_embed, wflat)
    preds = pl.pallas_call(
        _matmul_body,
        grid=(NBLK,),
        in_specs=[pl.BlockSpec((RBLK, D), lambda i: (i, 0)),
                  pl.BlockSpec((B, D), lambda i: (0, 0))],
        out_specs=pl.BlockSpec((RBLK, B), lambda i: (i, 0)),
        out_shape=jax.ShapeDtypeStruct((B, B), jnp.float32),
    )(u_e, picomb)
    return (u_e, picomb, nicomb, h_e, r_e, pt_e, nt_e, preds)
